# baseline (speedup 1.0000x reference)
import functools

import jax
import jax.numpy as jnp
from jax import lax
from jax.experimental import pallas as pl
from jax.experimental.pallas import tpu as pltpu
from jax.experimental.pallas import tpu_sc as plsc

G, N, K = 2048, 40, 8
C_IN, C_OUT = 32, 6
H = W = 512
D_IN = 576

GB = 64
SB = 128
HWB = (H * W) // (G // GB)

NW = 32
SC_G = 1024
TC_G = G - SC_G
GPW = SC_G // NW
CHUNK = 4
NCH = D_IN // 16


def _sc_stats_body(x_hbm, out_hbm, buf, acc, sem):
    wid = lax.axis_index("s") * 2 + lax.axis_index("c")
    g0 = TC_G + wid * GPW
    zero = jnp.zeros((16,), jnp.float32)
    for j in range(NCH):
        acc[0, pl.ds(j * 16, 16)] = zero
        acc[1, pl.ds(j * 16, 16)] = zero

    def chunk_body(c, carry):
        pltpu.async_copy(x_hbm.at[pl.ds(g0 + c * CHUNK, CHUNK)], buf, sem).wait()

        def graph_body(cg, carry2):
            def row_body(rn, carry3):
                for j in range(NCH):
                    v = buf[cg, rn, pl.ds(j * 16, 16)]
                    plsc.addupdate(acc.at[0, pl.ds(j * 16, 16)], v)
                    plsc.addupdate(acc.at[1, pl.ds(j * 16, 16)], v * v)
                return carry3

            return lax.fori_loop(0, N, row_body, carry2)

        return lax.fori_loop(0, CHUNK, graph_body, carry)

    lax.fori_loop(0, GPW // CHUNK, chunk_body, 0)
    pltpu.sync_copy(acc, out_hbm.at[wid])


def _sc_stats(node_feats):
    mesh = plsc.VectorSubcoreMesh(core_axis_name="c", subcore_axis_name="s")
    fn = functools.partial(
        pl.kernel,
        mesh=mesh,
        out_type=jax.ShapeDtypeStruct((NW, 2, D_IN), jnp.float32),
        scratch_types=[
            pltpu.VMEM((CHUNK, N, D_IN), jnp.float32),
            pltpu.VMEM((2, D_IN), jnp.float32),
            pltpu.SemaphoreType.DMA,
        ],
    )(_sc_stats_body)
    return fn(node_feats)


def _stats_kernel(x_ref, out_ref):
    i = pl.program_id(0)
    x = x_ref[...]
    s = jnp.sum(x, axis=(0, 1))
    s2 = jnp.sum(x * x, axis=(0, 1))
    part = jnp.stack([s, s2], axis=0)

    @pl.when(i == 0)
    def _():
        out_ref[...] = jnp.zeros_like(out_ref)

    out_ref[...] += part


def _layer(xb, a_blk, w_ref, b_ref, d_in, f_out):
    pa = jnp.dot(xb, w_ref[:d_in, :], preferred_element_type=jnp.float32)
    pb = jnp.dot(xb, w_ref[d_in:, :], preferred_element_type=jnp.float32)
    pb3 = pb.reshape(GB, N, f_out)
    agg = jax.lax.dot_general(
        a_blk, pb3.astype(jnp.bfloat16),
        dimension_numbers=(((2,), (1,)), ((0,), (0,))),
        preferred_element_type=jnp.float32)
    h = pa.reshape(GB, N, f_out) + agg + b_ref[...]
    return jnp.maximum(h, 0.0).reshape(GB * N, f_out).astype(jnp.bfloat16)


def _gcn_kernel(x_ref, a_ref, knn_ref, stats_ref, scstats_ref,
                w1_ref, b1_ref, w2_ref, b2_ref, w3_ref, b3_ref, w4_ref, b4_ref,
                wc1_ref, bc1_ref, pa_ref, wc2_ref, bc2_ref,
                img_ref, cw_ref, cb_ref,
                out_ref, pred_ref):
    pred_ref[...] = (jnp.dot(cw_ref[...], img_ref[...],
                             preferred_element_type=jnp.float32) + cb_ref[...])

    total = float(G * N)
    sums = stats_ref[...] + jnp.sum(scstats_ref[...], axis=0)
    mean = sums[0, :] / total
    var = sums[1, :] / total - mean * mean
    rinv = jax.lax.rsqrt(var + 1e-5)

    x = ((x_ref[...] - mean) * rinv).astype(jnp.bfloat16)
    a_blk = a_ref[...].astype(jnp.bfloat16)

    xf = x.reshape(GB * N, D_IN)
    xf = _layer(xf, a_blk, w1_ref, b1_ref, D_IN, 512)
    xf = _layer(xf, a_blk, w2_ref, b2_ref, 512, 256)
    xf = _layer(xf, a_blk, w3_ref, b3_ref, 256, 128)
    xf = _layer(xf, a_blk, w4_ref, b4_ref, 128, 64)

    h = jnp.dot(xf, wc1_ref[...], preferred_element_type=jnp.float32) + bc1_ref[...]
    h = jnp.where(h >= 0, h, pa_ref[...] * h)
    p = (jnp.dot(h, wc2_ref[...], preferred_element_type=jnp.float32)
         + bc2_ref[...])
    p3 = p.reshape(GB, N, 2)

    ids = knn_ref[...]
    iota_n = jax.lax.broadcasted_iota(jnp.int32, (GB, N), 1)
    edges = []
    for k in range(K):
        mask = (iota_n == ids[:, k][:, None]).astype(jnp.float32)
        edges.append(jnp.sum(mask[:, :, None] * p3, axis=1))
    out_ref[...] = jnp.stack(edges, axis=1).reshape(GB * K, 2)


def kernel(inputs, node_feats, A, knn_inds, conv_w, conv_b,
           w1, b1, w2, b2, w3, b3, w4, b4, wc1, bc1, prelu_a, wc2, bc2):
    sc_part = _sc_stats(node_feats)
    tc_part = pl.pallas_call(
        _stats_kernel,
        grid=(TC_G // SB,),
        in_specs=[pl.BlockSpec((SB, N, D_IN), lambda i: (i, 0, 0))],
        out_specs=pl.BlockSpec((2, D_IN), lambda i: (0, 0)),
        out_shape=jax.ShapeDtypeStruct((2, D_IN), jnp.float32),
    )(node_feats)

    w1b = w1.astype(jnp.bfloat16)
    w2b = w2.astype(jnp.bfloat16)
    w3b = w3.astype(jnp.bfloat16)
    w4b = w4.astype(jnp.bfloat16)
    x2 = inputs.reshape(C_IN, H * W)
    const = lambda shape: pl.BlockSpec(shape, lambda i: tuple(0 for _ in shape))
    gcn_pred, pred = pl.pallas_call(
        _gcn_kernel,
        grid=(G // GB,),
        in_specs=[
            pl.BlockSpec((GB, N, D_IN), lambda i: (i, 0, 0)),
            pl.BlockSpec((GB, N, N), lambda i: (i, 0, 0)),
            pl.BlockSpec((GB, K), lambda i: (i, 0)),
            const((2, D_IN)),
            const((NW, 2, D_IN)),
            const((2 * D_IN, 512)), const((512,)),
            const((1024, 256)), const((256,)),
            const((512, 128)), const((128,)),
            const((256, 64)), const((64,)),
            const((64, 32)), const((32,)), const((32,)),
            const((32, 2)), const((2,)),
            pl.BlockSpec((C_IN, HWB), lambda i: (0, i)),
            const((C_OUT, C_IN)),
            const((C_OUT, 1)),
        ],
        out_specs=[
            pl.BlockSpec((GB * K, 2), lambda i: (i, 0)),
            pl.BlockSpec((C_OUT, HWB), lambda i: (0, i)),
        ],
        out_shape=[
            jax.ShapeDtypeStruct((G * K, 2), jnp.float32),
            jax.ShapeDtypeStruct((C_OUT, H * W), jnp.float32),
        ],
    )(node_feats, A, knn_inds, tc_part, sc_part,
      w1b, b1, w2b, b2, w3b, b3, w4b, b4, wc1, bc1, prelu_a, wc2, bc2,
      x2, conv_w, conv_b.reshape(C_OUT, 1))
    pred_maps = pred.reshape(1, C_OUT, H, W)

    return (pred_maps, gcn_pred)


# speedup vs baseline: 1.2388x; 1.2388x over previous
import jax
import jax.numpy as jnp
from jax.experimental import pallas as pl

G, N, K = 2048, 40, 8
C_IN, C_OUT = 32, 6
H = W = 512
D_IN = 576

GB = 64
SB = 128
HWB = (H * W) // (G // GB)

def _stats_kernel(x_ref, out_ref):
    i = pl.program_id(0)
    x = x_ref[...]
    s = jnp.sum(x, axis=(0, 1))
    s2 = jnp.sum(x * x, axis=(0, 1))
    part = jnp.stack([s, s2], axis=0)

    @pl.when(i == 0)
    def _():
        out_ref[...] = jnp.zeros_like(out_ref)

    out_ref[...] += part


def _layer(xb, a_blk, w_ref, b_ref, d_in, f_out):
    pa = jnp.dot(xb, w_ref[:d_in, :], preferred_element_type=jnp.float32)
    pb = jnp.dot(xb, w_ref[d_in:, :], preferred_element_type=jnp.float32)
    pb3 = pb.reshape(GB, N, f_out)
    agg = jax.lax.dot_general(
        a_blk, pb3.astype(jnp.bfloat16),
        dimension_numbers=(((2,), (1,)), ((0,), (0,))),
        preferred_element_type=jnp.float32)
    h = pa.reshape(GB, N, f_out) + agg + b_ref[...]
    return jnp.maximum(h, 0.0).reshape(GB * N, f_out).astype(jnp.bfloat16)


def _gcn_kernel(x_ref, a_ref, knn_ref, stats_ref,
                w1_ref, b1_ref, w2_ref, b2_ref, w3_ref, b3_ref, w4_ref, b4_ref,
                wc1_ref, bc1_ref, pa_ref, wc2_ref, bc2_ref,
                img_ref, cw_ref, cb_ref,
                out_ref, pred_ref):
    pred_ref[...] = (jnp.dot(cw_ref[...], img_ref[...],
                             preferred_element_type=jnp.float32) + cb_ref[...])

    total = float(G * N)
    mean = stats_ref[0, :] / total
    var = stats_ref[1, :] / total - mean * mean
    rinv = jax.lax.rsqrt(var + 1e-5)

    x = ((x_ref[...] - mean) * rinv).astype(jnp.bfloat16)
    a_blk = a_ref[...].astype(jnp.bfloat16)

    xf = x.reshape(GB * N, D_IN)
    xf = _layer(xf, a_blk, w1_ref, b1_ref, D_IN, 512)
    xf = _layer(xf, a_blk, w2_ref, b2_ref, 512, 256)
    xf = _layer(xf, a_blk, w3_ref, b3_ref, 256, 128)
    xf = _layer(xf, a_blk, w4_ref, b4_ref, 128, 64)

    h = jnp.dot(xf, wc1_ref[...], preferred_element_type=jnp.float32) + bc1_ref[...]
    h = jnp.where(h >= 0, h, pa_ref[...] * h)
    p = (jnp.dot(h, wc2_ref[...], preferred_element_type=jnp.float32)
         + bc2_ref[...])
    p3 = p.reshape(GB, N, 2)

    ids = knn_ref[...]
    iota_n = jax.lax.broadcasted_iota(jnp.int32, (GB, N), 1)
    edges = []
    for k in range(K):
        mask = (iota_n == ids[:, k][:, None]).astype(jnp.float32)
        edges.append(jnp.sum(mask[:, :, None] * p3, axis=1))
    out_ref[...] = jnp.stack(edges, axis=1).reshape(GB * K, 2)


def kernel(inputs, node_feats, A, knn_inds, conv_w, conv_b,
           w1, b1, w2, b2, w3, b3, w4, b4, wc1, bc1, prelu_a, wc2, bc2):
    stats = pl.pallas_call(
        _stats_kernel,
        grid=(G // SB,),
        in_specs=[pl.BlockSpec((SB, N, D_IN), lambda i: (i, 0, 0))],
        out_specs=pl.BlockSpec((2, D_IN), lambda i: (0, 0)),
        out_shape=jax.ShapeDtypeStruct((2, D_IN), jnp.float32),
    )(node_feats)

    w1b = w1.astype(jnp.bfloat16)
    w2b = w2.astype(jnp.bfloat16)
    w3b = w3.astype(jnp.bfloat16)
    w4b = w4.astype(jnp.bfloat16)
    x2 = inputs.reshape(C_IN, H * W)
    const = lambda shape: pl.BlockSpec(shape, lambda i: tuple(0 for _ in shape))
    gcn_pred, pred = pl.pallas_call(
        _gcn_kernel,
        grid=(G // GB,),
        in_specs=[
            pl.BlockSpec((GB, N, D_IN), lambda i: (i, 0, 0)),
            pl.BlockSpec((GB, N, N), lambda i: (i, 0, 0)),
            pl.BlockSpec((GB, K), lambda i: (i, 0)),
            const((2, D_IN)),
            const((2 * D_IN, 512)), const((512,)),
            const((1024, 256)), const((256,)),
            const((512, 128)), const((128,)),
            const((256, 64)), const((64,)),
            const((64, 32)), const((32,)), const((32,)),
            const((32, 2)), const((2,)),
            pl.BlockSpec((C_IN, HWB), lambda i: (0, i)),
            const((C_OUT, C_IN)),
            const((C_OUT, 1)),
        ],
        out_specs=[
            pl.BlockSpec((GB * K, 2), lambda i: (i, 0)),
            pl.BlockSpec((C_OUT, HWB), lambda i: (0, i)),
        ],
        out_shape=[
            jax.ShapeDtypeStruct((G * K, 2), jnp.float32),
            jax.ShapeDtypeStruct((C_OUT, H * W), jnp.float32),
        ],
    )(node_feats, A, knn_inds, stats,
      w1b, b1, w2b, b2, w3b, b3, w4b, b4, wc1, bc1, prelu_a, wc2, bc2,
      x2, conv_w, conv_b.reshape(C_OUT, 1))
    pred_maps = pred.reshape(1, C_OUT, H, W)

    return (pred_maps, gcn_pred)
